# Initial kernel scaffold; baseline (speedup 1.0000x reference)
import jax
import jax.numpy as jnp
from jax.experimental import pallas as pl


def kernel(x, edge_index, edge_attr, embed, W1, b1, g1, be1, W2, b2, g2, be2, W3, b3, g3, be3):
    raise NotImplementedError("write your pallas kernel here")


# speedup vs baseline: 2.7705x; 2.7705x over previous
import functools

import jax
import jax.numpy as jnp
from jax import lax
from jax.experimental import pallas as pl
from jax.experimental.pallas import tpu as pltpu
from jax.experimental.pallas import tpu_sc as plsc

N = 10000
E = 160000
H = 512
D_IN = 255
NC, NS = 2, 16
NW = NC * NS
GCH = 128
GPW = 39
GTAIL = NW * GPW * GCH
SCH = 80
SPT = 125
CG = 128
NG = H // CG
NPT = 624
BN = 1000
BE = 800


def _ln(h, g, b):
    mu = jnp.mean(h, axis=-1, keepdims=True)
    var = jnp.mean((h - mu) ** 2, axis=-1, keepdims=True)
    return (h - mu) * lax.rsqrt(var + 1e-5) * g + b


def _node_body(x_ref, w1a_ref, embed_ref, w1bT_ref, b1_ref, p_ref):
    xb = x_ref[...]
    q = jnp.dot(embed_ref[...], w1bT_ref[...],
                preferred_element_type=jnp.float32)
    et = xb[:, 255].astype(jnp.int32).reshape(BN, 1)
    onehot = (et == lax.broadcasted_iota(jnp.int32, (1, 8), 1)).astype(jnp.float32)
    p = jnp.dot(xb, w1a_ref[...], preferred_element_type=jnp.float32)
    p = p + jnp.dot(onehot, q, preferred_element_type=jnp.float32)
    p_ref[...] = p + b1_ref[0:1, :]


def _mlp_body(g_ref, attr_ref, w1cT_ref, w2T_ref, w3T_ref, vecs_ref, h3_ref):
    pre1 = g_ref[...] + jnp.dot(attr_ref[...], w1cT_ref[...],
                                preferred_element_type=jnp.float32)
    h = _ln(jax.nn.relu(pre1), vecs_ref[0:1, :], vecs_ref[1:2, :])
    pre2 = jnp.dot(h, w2T_ref[...], preferred_element_type=jnp.float32) + vecs_ref[2:3, :]
    h = _ln(jax.nn.relu(pre2), vecs_ref[3:4, :], vecs_ref[4:5, :])
    pre3 = jnp.dot(h, w3T_ref[...], preferred_element_type=jnp.float32) + vecs_ref[5:6, :]
    h3_ref[...] = _ln(jax.nn.relu(pre3), vecs_ref[6:7, :], vecs_ref[7:8, :])


_sc_mesh = plsc.VectorSubcoreMesh(core_axis_name="c", subcore_axis_name="s")


@functools.partial(
    pl.kernel,
    out_type=jax.ShapeDtypeStruct((E, H), jnp.float32),
    mesh=_sc_mesh,
    scratch_types=[
        pltpu.VMEM((GCH,), jnp.int32),
        pltpu.VMEM((GCH, H), jnp.float32),
        pltpu.SemaphoreType.DMA,
    ],
)
def _gather(p_hbm, src_hbm, g_hbm, idx_v, buf_v, sem):
    c = lax.axis_index("c")
    s = lax.axis_index("s")
    wid = c * NS + s

    def chunk(base):
        pltpu.sync_copy(src_hbm.at[pl.ds(base, GCH)], idx_v)
        pltpu.async_copy(p_hbm.at[idx_v], buf_v, sem).wait()
        pltpu.sync_copy(buf_v, g_hbm.at[pl.ds(base, GCH)])

    def body(i, carry):
        chunk(wid * GPW * GCH + i * GCH)
        return carry

    lax.fori_loop(0, GPW, body, 0)

    @pl.when(wid < 2)
    def _():
        chunk(GTAIL + wid * GCH)


@functools.partial(
    pl.kernel,
    out_type=jax.ShapeDtypeStruct((N, H), jnp.float32),
    mesh=_sc_mesh,
    scratch_types=[
        pltpu.VMEM((SCH,), jnp.int32),
        pltpu.VMEM((SCH, CG), jnp.float32),
        pltpu.VMEM_SHARED((N, CG), jnp.float32),
    ],
)
def _scatter(h3_hbm, dst_hbm, zeros_hbm, out_hbm, idx_v, dat_v, acc_sh):
    c = lax.axis_index("c")
    s = lax.axis_index("s")
    last = NS * NPT
    for gi in range(2):
        col0 = (c * 2 + gi) * CG
        pltpu.sync_copy(zeros_hbm.at[pl.ds(s * NPT, NPT)],
                        acc_sh.at[pl.ds(s * NPT, NPT)])

        @pl.when(s == NS - 1)
        def _():
            pltpu.sync_copy(zeros_hbm.at[pl.ds(last, N - last)],
                            acc_sh.at[pl.ds(last, N - last)])

        plsc.subcore_barrier()

        def body(j, carry):
            base = s * SPT * SCH + j * SCH
            pltpu.sync_copy(dst_hbm.at[pl.ds(base, SCH)], idx_v)
            pltpu.sync_copy(h3_hbm.at[pl.ds(base, SCH), pl.ds(col0, CG)], dat_v)
            pltpu.sync_copy(dat_v, acc_sh.at[idx_v], add=True)
            return carry

        lax.fori_loop(0, SPT, body, 0)
        plsc.subcore_barrier()
        pltpu.sync_copy(acc_sh.at[pl.ds(s * NPT, NPT)],
                        out_hbm.at[pl.ds(s * NPT, NPT), pl.ds(col0, CG)])

        @pl.when(s == NS - 1)
        def _():
            pltpu.sync_copy(acc_sh.at[pl.ds(last, N - last)],
                            out_hbm.at[pl.ds(last, N - last), pl.ds(col0, CG)])

        plsc.subcore_barrier()


def kernel(x, edge_index, edge_attr, embed,
           W1, b1, g1, be1, W2, b2, g2, be2, W3, b3, g3, be3):
    src = edge_index[0]
    dst = edge_index[1]

    w1aT = jnp.pad(W1[:, :D_IN].T, ((0, 1), (0, 0)))
    w1bT = W1[:, D_IN:D_IN + 32].T
    w1cT = W1[:, D_IN + 32:].T
    b1b = jnp.broadcast_to(b1, (8, H))
    vecs = jnp.stack([g1, be1, b2, g2, be2, b3, g3, be3])

    p = pl.pallas_call(
        _node_body,
        grid=(N // BN,),
        in_specs=[
            pl.BlockSpec((BN, 256), lambda i: (i, 0)),
            pl.BlockSpec((256, H), lambda i: (0, 0)),
            pl.BlockSpec((8, 32), lambda i: (0, 0)),
            pl.BlockSpec((32, H), lambda i: (0, 0)),
            pl.BlockSpec((8, H), lambda i: (0, 0)),
        ],
        out_specs=pl.BlockSpec((BN, H), lambda i: (i, 0)),
        out_shape=jax.ShapeDtypeStruct((N, H), jnp.float32),
    )(x, w1aT, embed, w1bT, b1b)

    g2d = _gather(p, src)

    h3 = pl.pallas_call(
        _mlp_body,
        grid=(E // BE,),
        in_specs=[
            pl.BlockSpec((BE, H), lambda i: (i, 0)),
            pl.BlockSpec((BE, 16), lambda i: (i, 0)),
            pl.BlockSpec((16, H), lambda i: (0, 0)),
            pl.BlockSpec((H, H), lambda i: (0, 0)),
            pl.BlockSpec((H, H), lambda i: (0, 0)),
            pl.BlockSpec((8, H), lambda i: (0, 0)),
        ],
        out_specs=pl.BlockSpec((BE, H), lambda i: (i, 0)),
        out_shape=jax.ShapeDtypeStruct((E, H), jnp.float32),
    )(g2d, edge_attr, w1cT, W2.T, W3.T, vecs)

    zeros = jnp.zeros((N, CG), jnp.float32)
    out = _scatter(h3, dst, zeros)
    return out
